# baseline (speedup 1.0000x reference)
import functools

import jax
import jax.numpy as jnp
from jax.experimental import pallas as pl
from jax.experimental.pallas import tpu as pltpu

N_EXP = 8
TOP_K = 2
D = 1024
DFF = 4096

TB = 256
TM = 256
BF = 512
NJ = DFF // BF
STEP_PAD = 128


def _fiota(shape, dim):
    return jax.lax.broadcasted_iota(jnp.int32, shape, dim).astype(jnp.float32)


def _lane_cumsum_excl(v):
    l = v.shape[1]
    acc = v
    sh = 1
    while sh < l:
        shifted = jnp.concatenate(
            [jnp.zeros((1, sh), jnp.float32), acc[:, :l - sh]], axis=1)
        acc = acc + shifted
        sh *= 2
    return acc - v

def _router_body(x_ref, w_ref, o0_ref, o1_ref):
    x = x_ref[...]
    w = w_ref[...]
    logits = jax.lax.dot_general(x, w, (((1,), (1,)), ((), ())),
                                 preferred_element_type=jnp.float32)
    m = jnp.max(logits, axis=1, keepdims=True)
    p = jnp.exp(logits - m)
    probs = p / jnp.sum(p, axis=1, keepdims=True)
    lane = _fiota((TB, N_EXP), 1)
    v1 = jnp.max(probs, axis=1, keepdims=True)
    idx0 = jnp.min(jnp.where(probs == v1, lane, float(N_EXP)), axis=1,
                   keepdims=True)
    oh0 = (lane == idx0).astype(jnp.float32)
    probs2 = jnp.where(oh0 > 0, -1.0, probs)
    v2 = jnp.max(probs2, axis=1, keepdims=True)
    idx1 = jnp.min(jnp.where(probs2 == v2, lane, float(N_EXP)), axis=1,
                   keepdims=True)
    oh1 = (lane == idx1).astype(jnp.float32)
    denom = v1 + v2 + 1e-9
    o0_ref[0] = oh0 * (v1 / denom)
    o1_ref[0] = oh1 * (v2 / denom)


def _run_router(x_flat, router_W):
    nt = x_flat.shape[0] // TB
    return pl.pallas_call(
        _router_body,
        grid=(nt,),
        in_specs=[
            pl.BlockSpec((TB, D), lambda t: (t, 0)),
            pl.BlockSpec((N_EXP, D), lambda t: (0, 0)),
        ],
        out_specs=[
            pl.BlockSpec((1, TB, N_EXP), lambda t: (t, 0, 0)),
            pl.BlockSpec((1, TB, N_EXP), lambda t: (t, 0, 0)),
        ],
        out_shape=[
            jax.ShapeDtypeStruct((nt, TB, N_EXP), jnp.float32),
            jax.ShapeDtypeStruct((nt, TB, N_EXP), jnp.float32),
        ],
    )(x_flat, router_W)


def _bookkeep_body(w0_ref, w1_ref, tok_ref, step_ref, base_ref):
    nt = w0_ref.shape[0]

    def loop_a(t, base):
        oh0 = (w0_ref[pl.ds(t, 1)].reshape(TB, N_EXP) > 0).astype(jnp.float32)
        oh1 = (w1_ref[pl.ds(t, 1)].reshape(TB, N_EXP) > 0).astype(jnp.float32)
        base_ref[pl.ds(t, 1), :] = base
        cnt = (jnp.sum(oh0, axis=0, keepdims=True)
               + jnp.sum(oh1, axis=0, keepdims=True))
        return base + cnt

    totals = jax.lax.fori_loop(0, nt, loop_a,
                               jnp.zeros((1, N_EXP), jnp.float32))

    off = _lane_cumsum_excl(totals)

    sa = _fiota((TB, TB), 0)
    sb = _fiota((TB, TB), 1)
    stri = (sb < sa).astype(jnp.float32)
    lane_tok = _fiota((TB, N_EXP), 1)

    def loop_b(t, carry):
        w0 = w0_ref[pl.ds(t, 1)].reshape(TB, N_EXP)
        w1 = w1_ref[pl.ds(t, 1)].reshape(TB, N_EXP)
        oh0 = (w0 > 0).astype(jnp.float32)
        oh1 = (w1 > 0).astype(jnp.float32)
        base = base_ref[pl.ds(t, 1), :]
        rank0 = jax.lax.dot_general(stri, oh0, (((1,), (0,)), ((), ())),
                                    preferred_element_type=jnp.float32)
        csum0 = jnp.sum(oh0, axis=0, keepdims=True)
        rank1 = csum0 + jax.lax.dot_general(stri, oh1,
                                            (((1,), (0,)), ((), ())),
                                            preferred_element_type=jnp.float32)
        g0 = off + base + rank0
        g1 = off + base + rank1
        p0 = jnp.sum(oh0 * g0, axis=1, keepdims=True)
        p1 = jnp.sum(oh1 * g1, axis=1, keepdims=True)
        w0s = jnp.sum(w0, axis=1, keepdims=True)
        w1s = jnp.sum(w1, axis=1, keepdims=True)
        tok_ref[pl.ds(t, 1)] = (p0 * (lane_tok == 0) + p1 * (lane_tok == 1)
                                + w0s * (lane_tok == 2)
                                + w1s * (lane_tok == 3)).reshape(1, TB, N_EXP)
        return carry

    jax.lax.fori_loop(0, nt, loop_b, jnp.float32(0.0))

    s = totals
    nonempty = s > 0
    f = jnp.floor(off / TM)
    last = jnp.floor((off + s - 1.0) / TM)
    cnt = jnp.where(nonempty, last - f + 1.0, 0.0)
    starts = _lane_cumsum_excl(cnt)
    ends = starts + cnt
    nstep = jnp.sum(cnt, axis=1, keepdims=True)

    ii = _fiota((STEP_PAD, 1), 0)
    e_i = jnp.sum((ii >= ends).astype(jnp.float32), axis=1, keepdims=True)
    e_i = jnp.minimum(e_i, float(N_EXP - 1))
    lane_s = _fiota((STEP_PAD, N_EXP), 1)
    ohe = (lane_s == e_i).astype(jnp.float32)
    f_i = jnp.sum(ohe * f, axis=1, keepdims=True)
    st_i = jnp.sum(ohe * starts, axis=1, keepdims=True)
    o_i = jnp.sum(ohe * off, axis=1, keepdims=True)
    s_i = jnp.sum(ohe * s, axis=1, keepdims=True)
    tid = f_i + (ii - st_i)
    lo = jnp.maximum(o_i, tid * TM)
    hi = jnp.minimum(o_i + s_i, tid * TM + TM)
    valid = ii < nstep
    last_tile = float(nt * TB * TOP_K // TM - 1)
    tid = jnp.where(valid, tid, last_tile)
    lo = jnp.where(valid, lo, 0.0)
    hi = jnp.where(valid, hi, 0.0)
    e_i = jnp.where(valid, e_i, float(N_EXP - 1))
    step_ref[...] = (tid * (lane_s == 0) + e_i * (lane_s == 1)
                     + lo * (lane_s == 2) + hi * (lane_s == 3))


def _run_bookkeep(w0, w1):
    nt = w0.shape[0]
    return pl.pallas_call(
        _bookkeep_body,
        grid=(1,),
        in_specs=[
            pl.BlockSpec((nt, TB, N_EXP), lambda i: (0, 0, 0)),
            pl.BlockSpec((nt, TB, N_EXP), lambda i: (0, 0, 0)),
        ],
        out_specs=[
            pl.BlockSpec((nt, TB, N_EXP), lambda i: (0, 0, 0)),
            pl.BlockSpec((STEP_PAD, N_EXP), lambda i: (0, 0)),
        ],
        out_shape=[
            jax.ShapeDtypeStruct((nt, TB, N_EXP), jnp.float32),
            jax.ShapeDtypeStruct((STEP_PAD, N_EXP), jnp.float32),
        ],
        scratch_shapes=[pltpu.VMEM((nt, N_EXP), jnp.float32)],
    )(w0, w1)


def _copy_body(src_ref, dst_ref, in_ref, out_ref):
    out_ref[...] = in_ref[...]


def _run_dispatch(x3, src, dst):
    ns = src.shape[0]
    grid_spec = pltpu.PrefetchScalarGridSpec(
        num_scalar_prefetch=2,
        grid=(ns,),
        in_specs=[pl.BlockSpec((1, 8, 128), lambda i, s, d: (s[i], 0, 0))],
        out_specs=pl.BlockSpec((1, 8, 128), lambda i, s, d: (d[i], 0, 0)),
    )
    return pl.pallas_call(
        _copy_body,
        grid_spec=grid_spec,
        out_shape=jax.ShapeDtypeStruct((ns, 8, 128), jnp.float32),
    )(src, dst, x3)


def _gmlp_body(tid_ref, eid_ref, lo_ref, hi_ref, x_ref, wfc_ref, bfc_ref,
               wp_ref, bp_ref, o_ref, acc_ref):
    i = pl.program_id(0)
    j = pl.program_id(1)
    lo = lo_ref[i]
    hi = hi_ref[i]

    @pl.when(lo < hi)
    def _():
        x = x_ref[...]
        wfc = wfc_ref[0]
        h = jax.lax.dot_general(x, wfc, (((1,), (1,)), ((), ())),
                                preferred_element_type=jnp.float32)
        h = h + bfc_ref[0, 0]
        h = 0.5 * h * (1.0 + jax.lax.erf(h * 0.7071067811865476))
        wp = wp_ref[0]
        y = jax.lax.dot_general(h, wp, (((1,), (1,)), ((), ())),
                                preferred_element_type=jnp.float32)

        @pl.when(j == 0)
        def _():
            acc_ref[...] = y + bp_ref[0]

        @pl.when(j > 0)
        def _():
            acc_ref[...] += y

        @pl.when(j == NJ - 1)
        def _():
            row = (tid_ref[i] * TM
                   + jax.lax.broadcasted_iota(jnp.int32, (TM, 1), 0))
            mask = (row >= lo) & (row < hi)
            o_ref[...] = jnp.where(mask, acc_ref[...], o_ref[...])


def _run_gmlp(x_sorted, W_fc, b_fc, W_proj, b_proj, tid, eid, lo, hi, nstep):
    ns = x_sorted.shape[0]
    grid_spec = pltpu.PrefetchScalarGridSpec(
        num_scalar_prefetch=4,
        grid=(nstep, NJ),
        in_specs=[
            pl.BlockSpec((TM, D), lambda i, j, t, e, l, h: (t[i], 0)),
            pl.BlockSpec((1, BF, D), lambda i, j, t, e, l, h: (e[i], j, 0)),
            pl.BlockSpec((1, 1, 1, BF),
                         lambda i, j, t, e, l, h: (e[i], j, 0, 0)),
            pl.BlockSpec((1, D, BF), lambda i, j, t, e, l, h: (e[i], 0, j)),
            pl.BlockSpec((1, 1, D), lambda i, j, t, e, l, h: (e[i], 0, 0)),
        ],
        out_specs=pl.BlockSpec((TM, D), lambda i, j, t, e, l, h: (t[i], 0)),
        scratch_shapes=[pltpu.VMEM((TM, D), jnp.float32)],
    )
    return pl.pallas_call(
        _gmlp_body,
        grid_spec=grid_spec,
        out_shape=jax.ShapeDtypeStruct((ns, D), jnp.float32),
    )(tid, eid, lo, hi, x_sorted,
      W_fc, b_fc.reshape(N_EXP, NJ, 1, BF), W_proj,
      b_proj.reshape(N_EXP, 1, D))


TPT = 4


def _combine_body(p0_ref, p1_ref, *refs):
    a = refs[:TPT]
    b = refs[TPT:2 * TPT]
    w0_ref = refs[2 * TPT]
    w1_ref = refs[2 * TPT + 1]
    out_ref = refs[2 * TPT + 2]
    i = pl.program_id(0)
    for r in range(TPT):
        t = i * TPT + r
        out_ref[r] = w0_ref[t] * a[r][0] + w1_ref[t] * b[r][0]


def _run_combine(y3, p0, p1, w0, w1):
    n = w0.shape[0]

    def mk0(r):
        return pl.BlockSpec((1, 8, 128),
                            lambda i, q0, q1, r=r: (q0[i * TPT + r], 0, 0))

    def mk1(r):
        return pl.BlockSpec((1, 8, 128),
                            lambda i, q0, q1, r=r: (q1[i * TPT + r], 0, 0))

    grid_spec = pltpu.PrefetchScalarGridSpec(
        num_scalar_prefetch=2,
        grid=(n // TPT,),
        in_specs=([mk0(r) for r in range(TPT)]
                  + [mk1(r) for r in range(TPT)]
                  + [pl.BlockSpec(memory_space=pltpu.SMEM)] * 2),
        out_specs=pl.BlockSpec((TPT, 8, 128),
                               lambda i, q0, q1: (i, 0, 0)),
    )
    return pl.pallas_call(
        _combine_body,
        grid_spec=grid_spec,
        out_shape=jax.ShapeDtypeStruct((n, 8, 128), jnp.float32),
    )(p0, p1, *([y3] * (2 * TPT)), w0, w1)


@jax.jit
def kernel(x, router_W, W_fc, b_fc, W_proj, b_proj):
    bsz, seq, _ = x.shape
    n = bsz * seq
    x_flat = x.reshape(n, D)

    w0, w1 = _run_router(x_flat, router_W)
    tok_meta, step_meta = _run_bookkeep(w0, w1)

    p0 = tok_meta[:, :, 0].reshape(n).astype(jnp.int32)
    p1 = tok_meta[:, :, 1].reshape(n).astype(jnp.int32)
    cw0 = tok_meta[:, :, 2].reshape(n)
    cw1 = tok_meta[:, :, 3].reshape(n)
    tid = step_meta[:, 0].astype(jnp.int32)
    eid = step_meta[:, 1].astype(jnp.int32)
    lo = step_meta[:, 2].astype(jnp.int32)
    hi = step_meta[:, 3].astype(jnp.int32)

    src = jnp.concatenate([jnp.arange(n, dtype=jnp.int32)] * 2)
    dst = jnp.concatenate([p0, p1])

    x3 = x_flat.reshape(n, 8, 128)
    xs3 = _run_dispatch(x3, src, dst)
    x_sorted = xs3.reshape(TOP_K * n, D)

    nstep = TOP_K * n // TM + N_EXP - 1
    y_sorted = _run_gmlp(x_sorted, W_fc, b_fc, W_proj, b_proj,
                         tid, eid, lo, hi, nstep)

    y3 = y_sorted.reshape(TOP_K * n, 8, 128)
    out3 = _run_combine(y3, p0, p1, cw0, cw1)
    return out3.reshape(bsz, seq, D)


# speedup vs baseline: 7.6459x; 7.6459x over previous
import functools

import jax
import jax.numpy as jnp
from jax.experimental import pallas as pl
from jax.experimental.pallas import tpu as pltpu
from jax.experimental.pallas import tpu_sc as plsc

N_EXP = 8
TOP_K = 2
D = 1024
DFF = 4096

TB = 256
TM = 256
BF = 512
NJ = DFF // BF
STEP_PAD = 128


def _fiota(shape, dim):
    return jax.lax.broadcasted_iota(jnp.int32, shape, dim).astype(jnp.float32)


def _lane_cumsum_excl(v):
    l = v.shape[1]
    acc = v
    sh = 1
    while sh < l:
        shifted = jnp.concatenate(
            [jnp.zeros((1, sh), jnp.float32), acc[:, :l - sh]], axis=1)
        acc = acc + shifted
        sh *= 2
    return acc - v

def _router_body(x_ref, w_ref, o0_ref, o1_ref):
    x = x_ref[...]
    w = w_ref[...]
    logits = jax.lax.dot_general(x, w, (((1,), (1,)), ((), ())),
                                 preferred_element_type=jnp.float32)
    m = jnp.max(logits, axis=1, keepdims=True)
    p = jnp.exp(logits - m)
    probs = p / jnp.sum(p, axis=1, keepdims=True)
    lane = _fiota((TB, N_EXP), 1)
    v1 = jnp.max(probs, axis=1, keepdims=True)
    idx0 = jnp.min(jnp.where(probs == v1, lane, float(N_EXP)), axis=1,
                   keepdims=True)
    oh0 = (lane == idx0).astype(jnp.float32)
    probs2 = jnp.where(oh0 > 0, -1.0, probs)
    v2 = jnp.max(probs2, axis=1, keepdims=True)
    idx1 = jnp.min(jnp.where(probs2 == v2, lane, float(N_EXP)), axis=1,
                   keepdims=True)
    oh1 = (lane == idx1).astype(jnp.float32)
    denom = v1 + v2 + 1e-9
    o0_ref[0] = oh0 * (v1 / denom)
    o1_ref[0] = oh1 * (v2 / denom)


def _run_router(x_flat, router_W):
    nt = x_flat.shape[0] // TB
    return pl.pallas_call(
        _router_body,
        grid=(nt,),
        in_specs=[
            pl.BlockSpec((TB, D), lambda t: (t, 0)),
            pl.BlockSpec((N_EXP, D), lambda t: (0, 0)),
        ],
        out_specs=[
            pl.BlockSpec((1, TB, N_EXP), lambda t: (t, 0, 0)),
            pl.BlockSpec((1, TB, N_EXP), lambda t: (t, 0, 0)),
        ],
        out_shape=[
            jax.ShapeDtypeStruct((nt, TB, N_EXP), jnp.float32),
            jax.ShapeDtypeStruct((nt, TB, N_EXP), jnp.float32),
        ],
    )(x_flat, router_W)


def _bookkeep_body(w0_ref, w1_ref, tok_ref, step_ref, base_ref):
    nt = w0_ref.shape[0]

    def loop_a(t, base):
        oh0 = (w0_ref[pl.ds(t, 1)].reshape(TB, N_EXP) > 0).astype(jnp.float32)
        oh1 = (w1_ref[pl.ds(t, 1)].reshape(TB, N_EXP) > 0).astype(jnp.float32)
        base_ref[pl.ds(t, 1), :] = base
        cnt = (jnp.sum(oh0, axis=0, keepdims=True)
               + jnp.sum(oh1, axis=0, keepdims=True))
        return base + cnt

    totals = jax.lax.fori_loop(0, nt, loop_a,
                               jnp.zeros((1, N_EXP), jnp.float32))

    off = _lane_cumsum_excl(totals)

    sa = _fiota((TB, TB), 0)
    sb = _fiota((TB, TB), 1)
    stri = (sb < sa).astype(jnp.float32)
    lane_tok = _fiota((TB, N_EXP), 1)

    def loop_b(t, carry):
        w0 = w0_ref[pl.ds(t, 1)].reshape(TB, N_EXP)
        w1 = w1_ref[pl.ds(t, 1)].reshape(TB, N_EXP)
        oh0 = (w0 > 0).astype(jnp.float32)
        oh1 = (w1 > 0).astype(jnp.float32)
        base = base_ref[pl.ds(t, 1), :]
        rank0 = jax.lax.dot_general(stri, oh0, (((1,), (0,)), ((), ())),
                                    preferred_element_type=jnp.float32)
        csum0 = jnp.sum(oh0, axis=0, keepdims=True)
        rank1 = csum0 + jax.lax.dot_general(stri, oh1,
                                            (((1,), (0,)), ((), ())),
                                            preferred_element_type=jnp.float32)
        g0 = off + base + rank0
        g1 = off + base + rank1
        p0 = jnp.sum(oh0 * g0, axis=1, keepdims=True)
        p1 = jnp.sum(oh1 * g1, axis=1, keepdims=True)
        w0s = jnp.sum(w0, axis=1, keepdims=True)
        w1s = jnp.sum(w1, axis=1, keepdims=True)
        tok_ref[pl.ds(t, 1)] = (p0 * (lane_tok == 0) + p1 * (lane_tok == 1)
                                + w0s * (lane_tok == 2)
                                + w1s * (lane_tok == 3)).reshape(1, TB, N_EXP)
        return carry

    jax.lax.fori_loop(0, nt, loop_b, jnp.float32(0.0))

    s = totals
    nonempty = s > 0
    f = jnp.floor(off / TM)
    last = jnp.floor((off + s - 1.0) / TM)
    cnt = jnp.where(nonempty, last - f + 1.0, 0.0)
    starts = _lane_cumsum_excl(cnt)
    ends = starts + cnt
    nstep = jnp.sum(cnt, axis=1, keepdims=True)

    ii = _fiota((STEP_PAD, 1), 0)
    e_i = jnp.sum((ii >= ends).astype(jnp.float32), axis=1, keepdims=True)
    e_i = jnp.minimum(e_i, float(N_EXP - 1))
    lane_s = _fiota((STEP_PAD, N_EXP), 1)
    ohe = (lane_s == e_i).astype(jnp.float32)
    f_i = jnp.sum(ohe * f, axis=1, keepdims=True)
    st_i = jnp.sum(ohe * starts, axis=1, keepdims=True)
    o_i = jnp.sum(ohe * off, axis=1, keepdims=True)
    s_i = jnp.sum(ohe * s, axis=1, keepdims=True)
    tid = f_i + (ii - st_i)
    lo = jnp.maximum(o_i, tid * TM)
    hi = jnp.minimum(o_i + s_i, tid * TM + TM)
    valid = ii < nstep
    last_tile = float(nt * TB * TOP_K // TM - 1)
    tid = jnp.where(valid, tid, last_tile)
    lo = jnp.where(valid, lo, 0.0)
    hi = jnp.where(valid, hi, 0.0)
    e_i = jnp.where(valid, e_i, float(N_EXP - 1))
    step_ref[...] = (tid * (lane_s == 0) + e_i * (lane_s == 1)
                     + lo * (lane_s == 2) + hi * (lane_s == 3))


def _run_bookkeep(w0, w1):
    nt = w0.shape[0]
    return pl.pallas_call(
        _bookkeep_body,
        grid=(1,),
        in_specs=[
            pl.BlockSpec((nt, TB, N_EXP), lambda i: (0, 0, 0)),
            pl.BlockSpec((nt, TB, N_EXP), lambda i: (0, 0, 0)),
        ],
        out_specs=[
            pl.BlockSpec((nt, TB, N_EXP), lambda i: (0, 0, 0)),
            pl.BlockSpec((STEP_PAD, N_EXP), lambda i: (0, 0)),
        ],
        out_shape=[
            jax.ShapeDtypeStruct((nt, TB, N_EXP), jnp.float32),
            jax.ShapeDtypeStruct((STEP_PAD, N_EXP), jnp.float32),
        ],
        scratch_shapes=[pltpu.VMEM((nt, N_EXP), jnp.float32)],
    )(w0, w1)


NW = 32
CH = 16
NCH = 8192 // NW // CH


def _run_dispatch(x_flat, p0r, p1r):
    n = x_flat.shape[0]
    mesh = plsc.VectorSubcoreMesh(core_axis_name="c", subcore_axis_name="s")

    @functools.partial(
        pl.kernel,
        out_type=jax.ShapeDtypeStruct((TOP_K * n, D), jnp.float32),
        mesh=mesh,
        scratch_types=[
            pltpu.VMEM((NCH, CH), jnp.int32),
            pltpu.VMEM((NCH, CH), jnp.int32),
            pltpu.VMEM((2, CH, D), jnp.float32),
            pltpu.SemaphoreType.DMA,
            pltpu.SemaphoreType.DMA,
        ],
    )
    def k(x_hbm, p0_hbm, p1_hbm, out_hbm, idx0, idx1, buf, sem0, sem1):
        wid = jax.lax.axis_index("s") * 2 + jax.lax.axis_index("c")
        pltpu.sync_copy(p0_hbm.at[wid], idx0)
        pltpu.sync_copy(p1_hbm.at[wid], idx1)
        base = wid * (NCH * CH)
        pending = [None, None]
        for c in range(NCH):
            b = c % 2
            if pending[b] is not None:
                pending[b][0].wait()
                pending[b][1].wait()
            pltpu.sync_copy(x_hbm.at[pl.ds(base + c * CH, CH)], buf.at[b])
            cp0 = pltpu.async_copy(buf.at[b], out_hbm.at[idx0.at[c]], sem0)
            cp1 = pltpu.async_copy(buf.at[b], out_hbm.at[idx1.at[c]], sem1)
            pending[b] = (cp0, cp1)
        for p in pending:
            if p is not None:
                p[0].wait()
                p[1].wait()

    return k(x_flat, p0r, p1r)


def _gmlp_body(tid_ref, eid_ref, lo_ref, hi_ref, x_ref, wfc_ref, bfc_ref,
               wp_ref, bp_ref, o_ref, acc_ref):
    i = pl.program_id(0)
    j = pl.program_id(1)
    lo = lo_ref[i]
    hi = hi_ref[i]

    @pl.when(lo < hi)
    def _():
        x = x_ref[...]
        wfc = wfc_ref[0]
        h = jax.lax.dot_general(x, wfc, (((1,), (1,)), ((), ())),
                                preferred_element_type=jnp.float32)
        h = h + bfc_ref[0, 0]
        h = 0.5 * h * (1.0 + jax.lax.erf(h * 0.7071067811865476))
        wp = wp_ref[0]
        y = jax.lax.dot_general(h, wp, (((1,), (1,)), ((), ())),
                                preferred_element_type=jnp.float32)

        @pl.when(j == 0)
        def _():
            acc_ref[...] = y + bp_ref[0]

        @pl.when(j > 0)
        def _():
            acc_ref[...] += y

        @pl.when(j == NJ - 1)
        def _():
            row = (tid_ref[i] * TM
                   + jax.lax.broadcasted_iota(jnp.int32, (TM, 1), 0))
            mask = (row >= lo) & (row < hi)
            o_ref[...] = jnp.where(mask, acc_ref[...], o_ref[...])


def _run_gmlp(x_sorted, W_fc, b_fc, W_proj, b_proj, tid, eid, lo, hi, nstep):
    ns = x_sorted.shape[0]
    grid_spec = pltpu.PrefetchScalarGridSpec(
        num_scalar_prefetch=4,
        grid=(nstep, NJ),
        in_specs=[
            pl.BlockSpec((TM, D), lambda i, j, t, e, l, h: (t[i], 0)),
            pl.BlockSpec((1, BF, D), lambda i, j, t, e, l, h: (e[i], j, 0)),
            pl.BlockSpec((1, 1, 1, BF),
                         lambda i, j, t, e, l, h: (e[i], j, 0, 0)),
            pl.BlockSpec((1, D, BF), lambda i, j, t, e, l, h: (e[i], 0, j)),
            pl.BlockSpec((1, 1, D), lambda i, j, t, e, l, h: (e[i], 0, 0)),
        ],
        out_specs=pl.BlockSpec((TM, D), lambda i, j, t, e, l, h: (t[i], 0)),
        scratch_shapes=[pltpu.VMEM((TM, D), jnp.float32)],
    )
    return pl.pallas_call(
        _gmlp_body,
        grid_spec=grid_spec,
        out_shape=jax.ShapeDtypeStruct((ns, D), jnp.float32),
    )(tid, eid, lo, hi, x_sorted,
      W_fc, b_fc.reshape(N_EXP, NJ, 1, BF), W_proj,
      b_proj.reshape(N_EXP, 1, D))


def _run_combine(y_sorted, p0r, p1r, w0r, w1r):
    n = w0r.shape[0] * w0r.shape[1]
    mesh = plsc.VectorSubcoreMesh(core_axis_name="c", subcore_axis_name="s")

    @functools.partial(
        pl.kernel,
        out_type=jax.ShapeDtypeStruct((n, D), jnp.float32),
        mesh=mesh,
        scratch_types=[
            pltpu.VMEM((NCH, CH), jnp.int32),
            pltpu.VMEM((NCH, CH), jnp.int32),
            pltpu.VMEM((NCH * CH, 16), jnp.float32),
            pltpu.VMEM((NCH * CH, 16), jnp.float32),
            pltpu.VMEM((CH, D), jnp.float32),
            pltpu.VMEM((CH, D), jnp.float32),
            pltpu.SemaphoreType.DMA,
            pltpu.SemaphoreType.DMA,
        ],
    )
    def k(y_hbm, p0_hbm, p1_hbm, w0_hbm, w1_hbm, out_hbm,
          idx0, idx1, w0v, w1v, buf0, buf1, sem0, sem1):
        wid = jax.lax.axis_index("s") * 2 + jax.lax.axis_index("c")
        pltpu.sync_copy(p0_hbm.at[wid], idx0)
        pltpu.sync_copy(p1_hbm.at[wid], idx1)
        pltpu.sync_copy(w0_hbm.at[wid], w0v)
        pltpu.sync_copy(w1_hbm.at[wid], w1v)
        base = wid * (NCH * CH)
        for c in range(NCH):
            g0 = pltpu.async_copy(y_hbm.at[idx0.at[c]], buf0, sem0)
            g1 = pltpu.async_copy(y_hbm.at[idx1.at[c]], buf1, sem1)
            g0.wait()
            g1.wait()

            def rbody(r, _, c=c):
                a0 = w0v[c * CH + r, :]
                a1 = w1v[c * CH + r, :]
                for l in range(D // 16):
                    sl = pl.ds(l * 16, 16)
                    buf0[r, sl] = a0 * buf0[r, sl] + a1 * buf1[r, sl]
                return 0

            jax.lax.fori_loop(0, CH, rbody, 0)
            pltpu.sync_copy(buf0, out_hbm.at[pl.ds(base + c * CH, CH)])

    return k(y_sorted, p0r, p1r, w0r, w1r)


@jax.jit
def kernel(x, router_W, W_fc, b_fc, W_proj, b_proj):
    bsz, seq, _ = x.shape
    n = bsz * seq
    x_flat = x.reshape(n, D)

    w0, w1 = _run_router(x_flat, router_W)
    tok_meta, step_meta = _run_bookkeep(w0, w1)

    p0 = tok_meta[:, :, 0].reshape(n).astype(jnp.int32)
    p1 = tok_meta[:, :, 1].reshape(n).astype(jnp.int32)
    cw0 = tok_meta[:, :, 2].reshape(n)
    cw1 = tok_meta[:, :, 3].reshape(n)
    tid = step_meta[:, 0].astype(jnp.int32)
    eid = step_meta[:, 1].astype(jnp.int32)
    lo = step_meta[:, 2].astype(jnp.int32)
    hi = step_meta[:, 3].astype(jnp.int32)

    p0r = p0.reshape(NW, NCH, CH)
    p1r = p1.reshape(NW, NCH, CH)
    x_sorted = _run_dispatch(x_flat, p0r, p1r)

    nstep = TOP_K * n // TM + N_EXP - 1
    y_sorted = _run_gmlp(x_sorted, W_fc, b_fc, W_proj, b_proj,
                         tid, eid, lo, hi, nstep)

    w0b = jnp.broadcast_to(cw0[:, None], (n, 16)).reshape(NW, NCH * CH, 16)
    w1b = jnp.broadcast_to(cw1[:, None], (n, 16)).reshape(NW, NCH * CH, 16)
    out = _run_combine(y_sorted, p0r, p1r, w0b, w1b)
    return out.reshape(bsz, seq, D)


# speedup vs baseline: 10.8234x; 1.4156x over previous
import functools

import jax
import jax.numpy as jnp
from jax.experimental import pallas as pl
from jax.experimental.pallas import tpu as pltpu
from jax.experimental.pallas import tpu_sc as plsc

N_EXP = 8
TOP_K = 2
D = 1024
DFF = 4096

TB = 256
TM = 512
BF = 512
NJ = DFF // BF
STEP_PAD = 128


def _fiota(shape, dim):
    return jax.lax.broadcasted_iota(jnp.int32, shape, dim).astype(jnp.float32)


def _lane_cumsum_excl(v):
    l = v.shape[1]
    acc = v
    sh = 1
    while sh < l:
        shifted = jnp.concatenate(
            [jnp.zeros((1, sh), jnp.float32), acc[:, :l - sh]], axis=1)
        acc = acc + shifted
        sh *= 2
    return acc - v

def _router_body(x_ref, w_ref, o0_ref, o1_ref):
    x = x_ref[...]
    w = w_ref[...]
    logits = jax.lax.dot_general(x, w, (((1,), (1,)), ((), ())),
                                 preferred_element_type=jnp.float32)
    m = jnp.max(logits, axis=1, keepdims=True)
    p = jnp.exp(logits - m)
    probs = p / jnp.sum(p, axis=1, keepdims=True)
    lane = _fiota((TB, N_EXP), 1)
    v1 = jnp.max(probs, axis=1, keepdims=True)
    idx0 = jnp.min(jnp.where(probs == v1, lane, float(N_EXP)), axis=1,
                   keepdims=True)
    oh0 = (lane == idx0).astype(jnp.float32)
    probs2 = jnp.where(oh0 > 0, -1.0, probs)
    v2 = jnp.max(probs2, axis=1, keepdims=True)
    idx1 = jnp.min(jnp.where(probs2 == v2, lane, float(N_EXP)), axis=1,
                   keepdims=True)
    oh1 = (lane == idx1).astype(jnp.float32)
    denom = v1 + v2 + 1e-9
    o0_ref[0] = oh0 * (v1 / denom)
    o1_ref[0] = oh1 * (v2 / denom)


def _run_router(x_flat, router_W):
    nt = x_flat.shape[0] // TB
    return pl.pallas_call(
        _router_body,
        grid=(nt,),
        in_specs=[
            pl.BlockSpec((TB, D), lambda t: (t, 0)),
            pl.BlockSpec((N_EXP, D), lambda t: (0, 0)),
        ],
        out_specs=[
            pl.BlockSpec((1, TB, N_EXP), lambda t: (t, 0, 0)),
            pl.BlockSpec((1, TB, N_EXP), lambda t: (t, 0, 0)),
        ],
        out_shape=[
            jax.ShapeDtypeStruct((nt, TB, N_EXP), jnp.float32),
            jax.ShapeDtypeStruct((nt, TB, N_EXP), jnp.float32),
        ],
    )(x_flat, router_W)


def _bookkeep_body(w0_ref, w1_ref, tok_ref, step_ref, base_ref):
    nt = w0_ref.shape[0]

    def loop_a(t, base):
        oh0 = (w0_ref[pl.ds(t, 1)].reshape(TB, N_EXP) > 0).astype(jnp.float32)
        oh1 = (w1_ref[pl.ds(t, 1)].reshape(TB, N_EXP) > 0).astype(jnp.float32)
        base_ref[pl.ds(t, 1), :] = base
        cnt = (jnp.sum(oh0, axis=0, keepdims=True)
               + jnp.sum(oh1, axis=0, keepdims=True))
        return base + cnt

    totals = jax.lax.fori_loop(0, nt, loop_a,
                               jnp.zeros((1, N_EXP), jnp.float32))

    off = _lane_cumsum_excl(totals)

    sa = _fiota((TB, TB), 0)
    sb = _fiota((TB, TB), 1)
    stri = (sb < sa).astype(jnp.float32)
    lane_tok = _fiota((TB, N_EXP), 1)

    def loop_b(t, carry):
        w0 = w0_ref[pl.ds(t, 1)].reshape(TB, N_EXP)
        w1 = w1_ref[pl.ds(t, 1)].reshape(TB, N_EXP)
        oh0 = (w0 > 0).astype(jnp.float32)
        oh1 = (w1 > 0).astype(jnp.float32)
        base = base_ref[pl.ds(t, 1), :]
        rank0 = jax.lax.dot_general(stri, oh0, (((1,), (0,)), ((), ())),
                                    preferred_element_type=jnp.float32)
        csum0 = jnp.sum(oh0, axis=0, keepdims=True)
        rank1 = csum0 + jax.lax.dot_general(stri, oh1,
                                            (((1,), (0,)), ((), ())),
                                            preferred_element_type=jnp.float32)
        g0 = off + base + rank0
        g1 = off + base + rank1
        p0 = jnp.sum(oh0 * g0, axis=1, keepdims=True)
        p1 = jnp.sum(oh1 * g1, axis=1, keepdims=True)
        w0s = jnp.sum(w0, axis=1, keepdims=True)
        w1s = jnp.sum(w1, axis=1, keepdims=True)
        tok_ref[pl.ds(t, 1)] = (p0 * (lane_tok == 0) + p1 * (lane_tok == 1)
                                + w0s * (lane_tok == 2)
                                + w1s * (lane_tok == 3)).reshape(1, TB, N_EXP)
        return carry

    jax.lax.fori_loop(0, nt, loop_b, jnp.float32(0.0))

    s = totals
    nonempty = s > 0
    f = jnp.floor(off / TM)
    last = jnp.floor((off + s - 1.0) / TM)
    cnt = jnp.where(nonempty, last - f + 1.0, 0.0)
    starts = _lane_cumsum_excl(cnt)
    ends = starts + cnt
    nstep = jnp.sum(cnt, axis=1, keepdims=True)

    ii = _fiota((STEP_PAD, 1), 0)
    e_i = jnp.sum((ii >= ends).astype(jnp.float32), axis=1, keepdims=True)
    e_i = jnp.minimum(e_i, float(N_EXP - 1))
    lane_s = _fiota((STEP_PAD, N_EXP), 1)
    ohe = (lane_s == e_i).astype(jnp.float32)
    f_i = jnp.sum(ohe * f, axis=1, keepdims=True)
    st_i = jnp.sum(ohe * starts, axis=1, keepdims=True)
    o_i = jnp.sum(ohe * off, axis=1, keepdims=True)
    s_i = jnp.sum(ohe * s, axis=1, keepdims=True)
    tid = f_i + (ii - st_i)
    lo = jnp.maximum(o_i, tid * TM)
    hi = jnp.minimum(o_i + s_i, tid * TM + TM)
    valid = ii < nstep
    last_tile = float(nt * TB * TOP_K // TM - 1)
    tid = jnp.where(valid, tid, last_tile)
    lo = jnp.where(valid, lo, 0.0)
    hi = jnp.where(valid, hi, 0.0)
    e_i = jnp.where(valid, e_i, float(N_EXP - 1))
    step_ref[...] = (tid * (lane_s == 0) + e_i * (lane_s == 1)
                     + lo * (lane_s == 2) + hi * (lane_s == 3))


def _run_bookkeep(w0, w1):
    nt = w0.shape[0]
    return pl.pallas_call(
        _bookkeep_body,
        grid=(1,),
        in_specs=[
            pl.BlockSpec((nt, TB, N_EXP), lambda i: (0, 0, 0)),
            pl.BlockSpec((nt, TB, N_EXP), lambda i: (0, 0, 0)),
        ],
        out_specs=[
            pl.BlockSpec((nt, TB, N_EXP), lambda i: (0, 0, 0)),
            pl.BlockSpec((STEP_PAD, N_EXP), lambda i: (0, 0)),
        ],
        out_shape=[
            jax.ShapeDtypeStruct((nt, TB, N_EXP), jnp.float32),
            jax.ShapeDtypeStruct((STEP_PAD, N_EXP), jnp.float32),
        ],
        scratch_shapes=[pltpu.VMEM((nt, N_EXP), jnp.float32)],
    )(w0, w1)


NW = 32
CH = 16
NCH = 8192 // NW // CH


def _run_dispatch(x_flat, p0r, p1r):
    n = x_flat.shape[0]
    mesh = plsc.VectorSubcoreMesh(core_axis_name="c", subcore_axis_name="s")

    @functools.partial(
        pl.kernel,
        out_type=jax.ShapeDtypeStruct((TOP_K * n, D), jnp.float32),
        mesh=mesh,
        scratch_types=[
            pltpu.VMEM((NCH, CH), jnp.int32),
            pltpu.VMEM((NCH, CH), jnp.int32),
            pltpu.VMEM((2, CH, D), jnp.float32),
            pltpu.SemaphoreType.DMA,
            pltpu.SemaphoreType.DMA,
        ],
    )
    def k(x_hbm, p0_hbm, p1_hbm, out_hbm, idx0, idx1, buf, sem0, sem1):
        wid = jax.lax.axis_index("s") * 2 + jax.lax.axis_index("c")
        pltpu.sync_copy(p0_hbm.at[wid], idx0)
        pltpu.sync_copy(p1_hbm.at[wid], idx1)
        base = wid * (NCH * CH)
        pending = [None, None]
        for c in range(NCH):
            b = c % 2
            if pending[b] is not None:
                pending[b][0].wait()
                pending[b][1].wait()
            pltpu.sync_copy(x_hbm.at[pl.ds(base + c * CH, CH)], buf.at[b])
            cp0 = pltpu.async_copy(buf.at[b], out_hbm.at[idx0.at[c]], sem0)
            cp1 = pltpu.async_copy(buf.at[b], out_hbm.at[idx1.at[c]], sem1)
            pending[b] = (cp0, cp1)
        for p in pending:
            if p is not None:
                p[0].wait()
                p[1].wait()

    return k(x_flat, p0r, p1r)


def _gmlp_body(tid_ref, eid_ref, lo_ref, hi_ref, x_ref, wfc_ref, bfc_ref,
               wp_ref, bp_ref, o_ref, acc_ref):
    i = pl.program_id(0)
    j = pl.program_id(1)
    lo = lo_ref[i]
    hi = hi_ref[i]

    @pl.when(lo < hi)
    def _():
        x = x_ref[...]
        wfc = wfc_ref[0]
        h = jax.lax.dot_general(x, wfc, (((1,), (1,)), ((), ())),
                                preferred_element_type=jnp.float32)
        h = h + bfc_ref[0, 0]
        h = 0.5 * h * (1.0 + jax.lax.erf(h * 0.7071067811865476))
        wp = wp_ref[0]
        y = jax.lax.dot_general(h, wp, (((1,), (1,)), ((), ())),
                                preferred_element_type=jnp.float32)

        @pl.when(j == 0)
        def _():
            acc_ref[...] = y + bp_ref[0]

        @pl.when(j > 0)
        def _():
            acc_ref[...] += y

        @pl.when(j == NJ - 1)
        def _():
            row = (tid_ref[i] * TM
                   + jax.lax.broadcasted_iota(jnp.int32, (TM, 1), 0))
            mask = (row >= lo) & (row < hi)
            o_ref[...] = jnp.where(mask, acc_ref[...], o_ref[...])


def _run_gmlp(x_sorted, W_fc, b_fc, W_proj, b_proj, tid, eid, lo, hi, nstep):
    ns = x_sorted.shape[0]
    grid_spec = pltpu.PrefetchScalarGridSpec(
        num_scalar_prefetch=4,
        grid=(nstep, NJ),
        in_specs=[
            pl.BlockSpec((TM, D), lambda i, j, t, e, l, h: (t[i], 0)),
            pl.BlockSpec((1, BF, D), lambda i, j, t, e, l, h: (e[i], j, 0)),
            pl.BlockSpec((1, 1, 1, BF),
                         lambda i, j, t, e, l, h: (e[i], j, 0, 0)),
            pl.BlockSpec((1, D, BF), lambda i, j, t, e, l, h: (e[i], 0, j)),
            pl.BlockSpec((1, 1, D), lambda i, j, t, e, l, h: (e[i], 0, 0)),
        ],
        out_specs=pl.BlockSpec((TM, D), lambda i, j, t, e, l, h: (t[i], 0)),
        scratch_shapes=[pltpu.VMEM((TM, D), jnp.float32)],
    )
    return pl.pallas_call(
        _gmlp_body,
        grid_spec=grid_spec,
        out_shape=jax.ShapeDtypeStruct((ns, D), jnp.float32),
    )(tid, eid, lo, hi, x_sorted,
      W_fc, b_fc.reshape(N_EXP, NJ, 1, BF), W_proj,
      b_proj.reshape(N_EXP, 1, D))


def _run_combine(y_sorted, p0r, p1r, w0r, w1r):
    n = w0r.shape[0] * w0r.shape[1]
    mesh = plsc.VectorSubcoreMesh(core_axis_name="c", subcore_axis_name="s")

    @functools.partial(
        pl.kernel,
        out_type=jax.ShapeDtypeStruct((n, D), jnp.float32),
        mesh=mesh,
        scratch_types=[
            pltpu.VMEM((NCH, CH), jnp.int32),
            pltpu.VMEM((NCH, CH), jnp.int32),
            pltpu.VMEM((NCH * CH, 16), jnp.float32),
            pltpu.VMEM((NCH * CH, 16), jnp.float32),
            pltpu.VMEM((CH, D), jnp.float32),
            pltpu.VMEM((CH, D), jnp.float32),
            pltpu.SemaphoreType.DMA,
            pltpu.SemaphoreType.DMA,
        ],
    )
    def k(y_hbm, p0_hbm, p1_hbm, w0_hbm, w1_hbm, out_hbm,
          idx0, idx1, w0v, w1v, buf0, buf1, sem0, sem1):
        wid = jax.lax.axis_index("s") * 2 + jax.lax.axis_index("c")
        pltpu.sync_copy(p0_hbm.at[wid], idx0)
        pltpu.sync_copy(p1_hbm.at[wid], idx1)
        pltpu.sync_copy(w0_hbm.at[wid], w0v)
        pltpu.sync_copy(w1_hbm.at[wid], w1v)
        base = wid * (NCH * CH)
        for c in range(NCH):
            g0 = pltpu.async_copy(y_hbm.at[idx0.at[c]], buf0, sem0)
            g1 = pltpu.async_copy(y_hbm.at[idx1.at[c]], buf1, sem1)
            g0.wait()
            g1.wait()

            def rbody(r, _, c=c):
                a0 = w0v[c * CH + r, :]
                a1 = w1v[c * CH + r, :]
                for l in range(D // 16):
                    sl = pl.ds(l * 16, 16)
                    buf0[r, sl] = a0 * buf0[r, sl] + a1 * buf1[r, sl]
                return 0

            jax.lax.fori_loop(0, CH, rbody, 0)
            pltpu.sync_copy(buf0, out_hbm.at[pl.ds(base + c * CH, CH)])

    return k(y_sorted, p0r, p1r, w0r, w1r)


@jax.jit
def kernel(x, router_W, W_fc, b_fc, W_proj, b_proj):
    bsz, seq, _ = x.shape
    n = bsz * seq
    x_flat = x.reshape(n, D)

    w0, w1 = _run_router(x_flat, router_W)
    tok_meta, step_meta = _run_bookkeep(w0, w1)

    p0 = tok_meta[:, :, 0].reshape(n).astype(jnp.int32)
    p1 = tok_meta[:, :, 1].reshape(n).astype(jnp.int32)
    cw0 = tok_meta[:, :, 2].reshape(n)
    cw1 = tok_meta[:, :, 3].reshape(n)
    tid = step_meta[:, 0].astype(jnp.int32)
    eid = step_meta[:, 1].astype(jnp.int32)
    lo = step_meta[:, 2].astype(jnp.int32)
    hi = step_meta[:, 3].astype(jnp.int32)

    p0r = p0.reshape(NW, NCH, CH)
    p1r = p1.reshape(NW, NCH, CH)
    x_sorted = _run_dispatch(x_flat, p0r, p1r)

    nstep = TOP_K * n // TM + N_EXP - 1
    y_sorted = _run_gmlp(x_sorted, W_fc, b_fc, W_proj, b_proj,
                         tid, eid, lo, hi, nstep)

    w0b = jnp.broadcast_to(cw0[:, None], (n, 16)).reshape(NW, NCH * CH, 16)
    w1b = jnp.broadcast_to(cw1[:, None], (n, 16)).reshape(NW, NCH * CH, 16)
    out = _run_combine(y_sorted, p0r, p1r, w0b, w1b)
    return out.reshape(bsz, seq, D)
